# baseline (speedup 1.0000x reference)
import jax
import jax.numpy as jnp
from jax.experimental import pallas as pl
from jax.experimental.pallas import tpu as pltpu


def _conv1_body(x_ref, w1_ref, b1_ref, o_ref):
    x = x_ref[0]
    t1 = jnp.stack([x[kh:kh + 126, kw:kw + 171]
                    for kh in range(3) for kw in range(3)], axis=0)
    w = w1_ref[...]
    wb = w.astype(jnp.bfloat16)
    wl = (w - wb.astype(jnp.float32)).astype(jnp.bfloat16)
    tb = t1.astype(jnp.bfloat16)
    tl = (t1 - tb.astype(jnp.float32)).astype(jnp.bfloat16)

    def _e(a, b):
        return jnp.einsum("ct,thw->chw", a, b,
                          preferred_element_type=jnp.float32)

    y = _e(wb, tb) + _e(wb, tl) + _e(wl, tb)
    y = jnp.maximum(y + b1_ref[...][:, :, None], 0.0)
    y = jnp.max(y.reshape(24, 63, 2, 171), axis=2)
    y = jnp.max(y[:, :, :168].reshape(24, 63, 42, 4), axis=3)
    o_ref[0] = y


def _conv_block(xin, w_ref, b_ref, ho, wo, cin, cout, pool):
    acc = jnp.zeros((ho * wo, cout), jnp.float32)
    for kh in range(3):
        for kw in range(3):
            p = xin[kh:kh + ho, kw:kw + wo, :].reshape(ho * wo, cin)
            acc = acc + jnp.dot(p, w_ref[kh * 3 + kw],
                                preferred_element_type=jnp.float32)
    y = jnp.maximum(acc + b_ref[...], 0.0).reshape(ho, wo, cout)
    if pool == 1:
        return y
    hp, wp = ho // 2, wo // 2
    y = y[:hp * 2, :wp * 2]
    y = jnp.max(y.reshape(hp, 2, wp * 2, cout), axis=1)
    y = jnp.max(y.reshape(hp, wp, 2, cout), axis=2)
    return y


def _convs_body(x_ref, w2_ref, b2_ref, w3_ref, b3_ref, w4_ref, b4_ref, o_ref):
    x1 = x_ref[0]
    x2 = _conv_block(x1, w2_ref, b2_ref, 61, 40, 24, 48, 2)
    x3 = _conv_block(x2, w3_ref, b3_ref, 28, 18, 48, 64, 2)
    x4 = _conv_block(x3, w4_ref, b4_ref, 12, 7, 64, 64, 1)
    o_ref[0] = x4


def _fc_softmax_body(x_ref, w_ref, b_ref, o_ref):
    logits = jnp.dot(x_ref[...], w_ref[...],
                     preferred_element_type=jnp.float32) + b_ref[...]
    m = jnp.max(logits, axis=0, keepdims=True)
    e = jnp.exp(logits - m)
    o_ref[...] = e / jnp.sum(e, axis=0, keepdims=True)


@jax.jit
def kernel(w1, s1, b1, w2, s2, b2, w3, s3, b3, w4, s4, b4, wfc, bfc, x):
    n = x.shape[0]
    x0 = x[:, 0].astype(jnp.float32)
    w1s = w1 * s1[:, None]
    b1c = b1[:, None]
    w2s = w2 * s2[None, :, :]
    w3s = w3 * s3[None, :, :]
    w4s = w4 * s4[None, :, :]
    a1 = pl.pallas_call(
        _conv1_body,
        out_shape=jax.ShapeDtypeStruct((n, 24, 63, 42), jnp.float32),
        grid=(n,),
        in_specs=[
            pl.BlockSpec((1, 128, 173), lambda i: (i, 0, 0)),
            pl.BlockSpec((24, 9), lambda i: (0, 0)),
            pl.BlockSpec((24, 1), lambda i: (0, 0)),
        ],
        out_specs=pl.BlockSpec((1, 24, 63, 42), lambda i: (i, 0, 0, 0)),
        compiler_params=pltpu.CompilerParams(
            dimension_semantics=("parallel",)),
    )(x0, w1s, b1c)
    x1 = jnp.transpose(a1, (0, 2, 3, 1))
    x4 = pl.pallas_call(
        _convs_body,
        out_shape=jax.ShapeDtypeStruct((n, 12, 7, 64), jnp.float32),
        grid=(n,),
        in_specs=[
            pl.BlockSpec((1, 63, 42, 24), lambda i: (i, 0, 0, 0)),
            pl.BlockSpec((9, 24, 48), lambda i: (0, 0, 0)),
            pl.BlockSpec((1, 48), lambda i: (0, 0)),
            pl.BlockSpec((9, 48, 64), lambda i: (0, 0, 0)),
            pl.BlockSpec((1, 64), lambda i: (0, 0)),
            pl.BlockSpec((9, 64, 64), lambda i: (0, 0, 0)),
            pl.BlockSpec((1, 64), lambda i: (0, 0)),
        ],
        out_specs=pl.BlockSpec((1, 12, 7, 64), lambda i: (i, 0, 0, 0)),
        compiler_params=pltpu.CompilerParams(
            dimension_semantics=("parallel",),
            vmem_limit_bytes=48 * 1024 * 1024),
    )(x1, w2s, b2, w3s, b3, w4s, b4)
    xf = x4.reshape(n, 12 * 7 * 64)
    return pl.pallas_call(
        _fc_softmax_body,
        out_shape=jax.ShapeDtypeStruct((n, 10), jnp.float32),
        grid=(1,),
        in_specs=[
            pl.BlockSpec((n, 5376), lambda i: (0, 0)),
            pl.BlockSpec((5376, 10), lambda i: (0, 0)),
            pl.BlockSpec((1, 10), lambda i: (0, 0)),
        ],
        out_specs=pl.BlockSpec((n, 10), lambda i: (0, 0)),
    )(xf, wfc, bfc)
